# baseline (speedup 1.0000x reference)
import functools

import jax
import jax.numpy as jnp
from jax import lax
from jax.experimental import pallas as pl
from jax.experimental.pallas import tpu as pltpu
from jax.experimental.pallas import tpu_sc as plsc

N_NODES = 10000
N_EDGES = 320000
D = 128
DE = 16
NC = 2
NS = 16
NW = NC * NS
EPW = N_EDGES // NW
C = 96
K = 105
EPW_P = K * C
N_PAD = 10240
RPT = N_PAD // NS
PAD_DST = N_NODES + 100

_MESH = plsc.VectorSubcoreMesh(core_axis_name="c", subcore_axis_name="s")


def _make_sc_a2():
    @functools.partial(
        pl.kernel,
        out_type=jax.ShapeDtypeStruct((NC, N_PAD, D), jnp.float32),
        mesh=_MESH,
        scratch_types=[
            [pltpu.VMEM((2, C), jnp.int32) for _ in range(3)],
            [pltpu.VMEM((C, D), jnp.float32) for _ in range(3)],
            pltpu.VMEM_SHARED((N_PAD, D), jnp.float32),
            [pltpu.SemaphoreType.DMA for _ in range(3)],
            [pltpu.SemaphoreType.DMA for _ in range(3)],
        ],
    )
    def sc_a2(x_hbm, fs_hbm, zeros_d_hbm, a2p_hbm,
              fs_v, rows_v, a2_sh, gsem, isem):
        cid = lax.axis_index("c")
        sid = lax.axis_index("s")
        wid = sid * NC + cid
        nbase = pl.multiple_of(sid * RPT, 8)
        pltpu.sync_copy(zeros_d_hbm, a2_sh.at[pl.ds(nbase, RPT)])
        pltpu.sync_copy(fs_hbm.at[wid, 0], fs_v[0])
        pltpu.async_copy(fs_hbm.at[wid, 1], fs_v[1], isem[1])
        pltpu.async_copy(fs_hbm.at[wid, 2], fs_v[2], isem[2])
        pltpu.async_copy(x_hbm.at[fs_v[0].at[1]], rows_v[0], gsem[0])
        pltpu.make_async_copy(fs_hbm.at[wid, 1], fs_v[1], isem[1]).wait()
        pltpu.async_copy(x_hbm.at[fs_v[1].at[1]], rows_v[1], gsem[1])
        plsc.subcore_barrier()

        def _step(j, b):
            pltpu.make_async_copy(x_hbm.at[fs_v[b].at[1]], rows_v[b],
                                  gsem[b]).wait()
            pltpu.sync_copy(rows_v[b], a2_sh.at[fs_v[b].at[0]], add=True)
            b2 = (b + 2) % 3

            @pl.when(j + 2 < K)
            def _():
                pltpu.make_async_copy(fs_hbm.at[wid, j + 2], fs_v[b2],
                                      isem[b2]).wait()
                pltpu.async_copy(x_hbm.at[fs_v[b2].at[1]], rows_v[b2],
                                 gsem[b2])

            @pl.when(j + 3 < K)
            def _():
                pltpu.async_copy(fs_hbm.at[wid, j + 3], fs_v[b], isem[b])

        @pl.loop(0, K // 3)
        def _triple(t):
            j = 3 * t
            _step(j, 0)
            _step(j + 1, 1)
            _step(j + 2, 2)

        plsc.subcore_barrier()
        pltpu.sync_copy(a2_sh.at[pl.ds(nbase, RPT)],
                        a2p_hbm.at[cid, pl.ds(nbase, RPT)])

    return sc_a2


DX = D


def _make_sc_a3cnt():
    @functools.partial(
        pl.kernel,
        out_type=jax.ShapeDtypeStruct((NC, N_PAD, DX), jnp.float32),
        mesh=_MESH,
        scratch_types=[
            pltpu.VMEM((K, C), jnp.int32),
            pltpu.VMEM((C, DX), jnp.float32),
            pltpu.VMEM((C, DX), jnp.float32),
            pltpu.VMEM_SHARED((N_PAD, DX), jnp.float32),
            pltpu.SemaphoreType.DMA,
            pltpu.SemaphoreType.DMA,
        ],
    )
    def sc_a3cnt(first_hbm, attrx_hbm, zeros_x_hbm, a3xp_hbm,
                 idx1_v, attrx0_v, attrx1_v, a3x_sh, sem0, sem1):
        cid = lax.axis_index("c")
        sid = lax.axis_index("s")
        wid = sid * NC + cid
        nbase = pl.multiple_of(sid * RPT, 8)
        pltpu.sync_copy(zeros_x_hbm, a3x_sh.at[pl.ds(nbase, RPT)])
        pltpu.sync_copy(first_hbm.at[wid], idx1_v)
        pltpu.async_copy(attrx_hbm.at[wid, 0], attrx0_v, sem0)
        plsc.subcore_barrier()

        @pl.loop(0, (K - 1) // 2)
        def _pair(p):
            j = 2 * p
            pltpu.make_async_copy(attrx_hbm.at[wid, j], attrx0_v, sem0).wait()
            pltpu.async_copy(attrx_hbm.at[wid, j + 1], attrx1_v, sem1)
            pltpu.sync_copy(attrx0_v, a3x_sh.at[idx1_v.at[j]], add=True)
            pltpu.make_async_copy(attrx_hbm.at[wid, j + 1], attrx1_v, sem1).wait()
            pltpu.async_copy(attrx_hbm.at[wid, j + 2], attrx0_v, sem0)
            pltpu.sync_copy(attrx1_v, a3x_sh.at[idx1_v.at[j + 1]], add=True)

        pltpu.make_async_copy(attrx_hbm.at[wid, K - 1], attrx0_v, sem0).wait()
        pltpu.sync_copy(attrx0_v, a3x_sh.at[idx1_v.at[K - 1]], add=True)

        plsc.subcore_barrier()
        pltpu.sync_copy(a3x_sh.at[pl.ds(nbase, RPT)],
                        a3xp_hbm.at[cid, pl.ds(nbase, RPT)])

    return sc_a3cnt


_SC_A2 = _make_sc_a2()
_SC_A3CNT = _make_sc_a3cnt()

_ROWS_PER_BLOCK = 400
_N_BLOCKS = N_NODES // _ROWS_PER_BLOCK


def _combine_body(x_ref, a2_ref, a3x_ref, we_ref, be_ref,
                  wn_ref, bn_ref, out_ref):
    f32 = jnp.float32
    x = x_ref[...]
    a2 = a2_ref[0] + a2_ref[1]
    a3x = a3x_ref[0] + a3x_ref[1]
    a3 = a3x[:, 0:DE]
    cnt = a3x[:, DE:DE + 1]
    ns = cnt * jnp.dot(x, we_ref[0:D, :], preferred_element_type=f32)
    ns = ns + jnp.dot(a2, we_ref[D:2 * D, :], preferred_element_type=f32)
    ns = ns + jnp.dot(a3, we_ref[2 * D:2 * D + DE, :], preferred_element_type=f32)
    ns = ns + cnt * be_ref[...]
    out = jnp.dot(x, wn_ref[0:D, :], preferred_element_type=f32)
    out = out + jnp.dot(ns, wn_ref[D:2 * D, :], preferred_element_type=f32)
    out_ref[...] = out + bn_ref[...]


def _combine(x, a2p, a3xp, W_edge, b_edge2, W_node, b_node2):
    r = _ROWS_PER_BLOCK
    return pl.pallas_call(
        _combine_body,
        grid=(_N_BLOCKS,),
        in_specs=[
            pl.BlockSpec((r, D), lambda i: (i, 0)),
            pl.BlockSpec((NC, r, D), lambda i: (0, i, 0)),
            pl.BlockSpec((NC, r, DX), lambda i: (0, i, 0)),
            pl.BlockSpec((2 * D + DE, D), lambda i: (0, 0)),
            pl.BlockSpec((1, D), lambda i: (0, 0)),
            pl.BlockSpec((2 * D, D), lambda i: (0, 0)),
            pl.BlockSpec((1, D), lambda i: (0, 0)),
        ],
        out_specs=pl.BlockSpec((r, D), lambda i: (i, 0)),
        out_shape=jax.ShapeDtypeStruct((N_NODES, D), jnp.float32),
    )(x, a2p, a3xp, W_edge, b_edge2, W_node, b_node2)


def kernel(node_features, edge_index, edge_attr, W_edge, b_edge, W_node, b_node):
    ei32 = edge_index.astype(jnp.int32)
    npad = EPW_P - EPW
    first_p = jnp.concatenate(
        [ei32[0].reshape(NW, EPW),
         jnp.full((NW, npad), PAD_DST, jnp.int32)], axis=1,
    ).reshape(NW, K, C)
    second_p = jnp.concatenate(
        [ei32[1].reshape(NW, EPW),
         jnp.zeros((NW, npad), jnp.int32)], axis=1,
    ).reshape(NW, K, C)
    fs = jnp.stack([first_p, second_p], axis=2)
    attrx = jnp.concatenate(
        [edge_attr, jnp.ones((N_EDGES, DX - DE), jnp.float32)], axis=1,
    ).reshape(NW, EPW, DX)
    attrx = jnp.concatenate(
        [attrx, jnp.zeros((NW, npad, DX), jnp.float32)], axis=1,
    ).reshape(NW, K, C, DX)
    zeros_d = jnp.zeros((RPT, D), jnp.float32)
    zeros_x = jnp.zeros((RPT, DX), jnp.float32)
    a2p = _SC_A2(node_features, fs, zeros_d)
    a3xp = _SC_A3CNT(first_p, attrx, zeros_x)
    nodes = _combine(node_features, a2p, a3xp, W_edge,
                     b_edge.reshape(1, D), W_node, b_node.reshape(1, D))
    return (nodes, edge_index, edge_attr)


# speedup vs baseline: 1.3876x; 1.3876x over previous
import functools

import jax
import jax.numpy as jnp
from jax import lax
from jax.experimental import pallas as pl
from jax.experimental.pallas import tpu as pltpu
from jax.experimental.pallas import tpu_sc as plsc

N_NODES = 10000
N_EDGES = 320000
D = 128
DE = 16
NC = 2
NS = 16
NW = NC * NS
EPW = N_EDGES // NW
C = 80
K = EPW // C
N_PAD = 10240
RPT = N_PAD // NS

_MESH = plsc.VectorSubcoreMesh(core_axis_name="c", subcore_axis_name="s")


def _make_sc_a2():
    @functools.partial(
        pl.kernel,
        out_type=jax.ShapeDtypeStruct((NC, N_PAD, D), jnp.float32),
        mesh=_MESH,
        scratch_types=[
            [pltpu.VMEM((2, C), jnp.int32) for _ in range(3)],
            [pltpu.VMEM((C, D), jnp.float32) for _ in range(3)],
            pltpu.VMEM_SHARED((N_PAD, D), jnp.float32),
            [pltpu.SemaphoreType.DMA for _ in range(3)],
            [pltpu.SemaphoreType.DMA for _ in range(3)],
        ],
    )
    def sc_a2(x_hbm, fs_hbm, zeros_d_hbm, a2p_hbm,
              fs_v, rows_v, a2_sh, gsem, isem):
        cid = lax.axis_index("c")
        sid = lax.axis_index("s")
        wid = sid * NC + cid
        nbase = pl.multiple_of(sid * RPT, 8)
        pltpu.sync_copy(zeros_d_hbm, a2_sh.at[pl.ds(nbase, RPT)])
        pltpu.sync_copy(fs_hbm.at[wid, 0], fs_v[0])
        pltpu.async_copy(fs_hbm.at[wid, 1], fs_v[1], isem[1])
        pltpu.async_copy(fs_hbm.at[wid, 2], fs_v[2], isem[2])
        pltpu.async_copy(x_hbm.at[fs_v[0].at[1]], rows_v[0], gsem[0])
        pltpu.make_async_copy(fs_hbm.at[wid, 1], fs_v[1], isem[1]).wait()
        pltpu.async_copy(x_hbm.at[fs_v[1].at[1]], rows_v[1], gsem[1])
        plsc.subcore_barrier()

        def _step(j, b):
            pltpu.make_async_copy(x_hbm.at[fs_v[b].at[1]], rows_v[b],
                                  gsem[b]).wait()
            b2 = (b + 2) % 3
            pltpu.make_async_copy(fs_hbm.at[wid, j + 2], fs_v[b2],
                                  isem[b2]).wait()
            pltpu.async_copy(x_hbm.at[fs_v[b2].at[1]], rows_v[b2], gsem[b2])
            pltpu.sync_copy(rows_v[b], a2_sh.at[fs_v[b].at[0]], add=True)

            @pl.when(j + 3 < K)
            def _():
                pltpu.async_copy(fs_hbm.at[wid, j + 3], fs_v[b], isem[b])

        @pl.loop(0, (K - 2) // 3)
        def _triple(t):
            j = 3 * t
            _step(j, 0)
            _step(j + 1, 1)
            _step(j + 2, 2)

        pltpu.make_async_copy(x_hbm.at[fs_v[0].at[1]], rows_v[0],
                              gsem[0]).wait()
        pltpu.sync_copy(rows_v[0], a2_sh.at[fs_v[0].at[0]], add=True)
        pltpu.make_async_copy(x_hbm.at[fs_v[1].at[1]], rows_v[1],
                              gsem[1]).wait()
        pltpu.sync_copy(rows_v[1], a2_sh.at[fs_v[1].at[0]], add=True)

        plsc.subcore_barrier()
        pltpu.sync_copy(a2_sh.at[pl.ds(nbase, RPT)],
                        a2p_hbm.at[cid, pl.ds(nbase, RPT)])

    return sc_a2


DX = D


def _make_sc_a3cnt():
    @functools.partial(
        pl.kernel,
        out_type=jax.ShapeDtypeStruct((NC, N_PAD, DX), jnp.float32),
        mesh=_MESH,
        scratch_types=[
            pltpu.VMEM((K, C), jnp.int32),
            pltpu.VMEM((C, DX), jnp.float32),
            pltpu.VMEM((C, DX), jnp.float32),
            pltpu.VMEM_SHARED((N_PAD, DX), jnp.float32),
            pltpu.SemaphoreType.DMA,
            pltpu.SemaphoreType.DMA,
        ],
    )
    def sc_a3cnt(first_hbm, attrx_hbm, zeros_x_hbm, a3xp_hbm,
                 idx1_v, attrx0_v, attrx1_v, a3x_sh, sem0, sem1):
        cid = lax.axis_index("c")
        sid = lax.axis_index("s")
        wid = sid * NC + cid
        nbase = pl.multiple_of(sid * RPT, 8)
        pltpu.sync_copy(zeros_x_hbm, a3x_sh.at[pl.ds(nbase, RPT)])
        pltpu.sync_copy(first_hbm.at[wid], idx1_v)
        pltpu.async_copy(attrx_hbm.at[wid, 0], attrx0_v, sem0)
        plsc.subcore_barrier()

        @pl.loop(0, (K - 1) // 2)
        def _pair(p):
            j = 2 * p
            pltpu.make_async_copy(attrx_hbm.at[wid, j], attrx0_v, sem0).wait()
            pltpu.async_copy(attrx_hbm.at[wid, j + 1], attrx1_v, sem1)
            pltpu.sync_copy(attrx0_v, a3x_sh.at[idx1_v.at[j]], add=True)
            pltpu.make_async_copy(attrx_hbm.at[wid, j + 1], attrx1_v, sem1).wait()
            pltpu.async_copy(attrx_hbm.at[wid, j + 2], attrx0_v, sem0)
            pltpu.sync_copy(attrx1_v, a3x_sh.at[idx1_v.at[j + 1]], add=True)

        pltpu.make_async_copy(attrx_hbm.at[wid, K - 1], attrx0_v, sem0).wait()
        pltpu.sync_copy(attrx0_v, a3x_sh.at[idx1_v.at[K - 1]], add=True)

        plsc.subcore_barrier()
        pltpu.sync_copy(a3x_sh.at[pl.ds(nbase, RPT)],
                        a3xp_hbm.at[cid, pl.ds(nbase, RPT)])

    return sc_a3cnt


_SC_A2 = _make_sc_a2()
_SC_A3CNT = _make_sc_a3cnt()

_ROWS_PER_BLOCK = 400
_N_BLOCKS = N_NODES // _ROWS_PER_BLOCK


def _combine_body(x_ref, a2_ref, a3x_ref, we_ref, be_ref,
                  wn_ref, bn_ref, out_ref):
    f32 = jnp.float32
    x = x_ref[...]
    a2 = a2_ref[0] + a2_ref[1]
    a3x = a3x_ref[0] + a3x_ref[1]
    a3 = a3x[:, 0:DE]
    cnt = a3x[:, DE:DE + 1]
    ns = cnt * jnp.dot(x, we_ref[0:D, :], preferred_element_type=f32)
    ns = ns + jnp.dot(a2, we_ref[D:2 * D, :], preferred_element_type=f32)
    ns = ns + jnp.dot(a3, we_ref[2 * D:2 * D + DE, :], preferred_element_type=f32)
    ns = ns + cnt * be_ref[...]
    out = jnp.dot(x, wn_ref[0:D, :], preferred_element_type=f32)
    out = out + jnp.dot(ns, wn_ref[D:2 * D, :], preferred_element_type=f32)
    out_ref[...] = out + bn_ref[...]


def _combine(x, a2p, a3xp, W_edge, b_edge2, W_node, b_node2):
    r = _ROWS_PER_BLOCK
    return pl.pallas_call(
        _combine_body,
        grid=(_N_BLOCKS,),
        in_specs=[
            pl.BlockSpec((r, D), lambda i: (i, 0)),
            pl.BlockSpec((NC, r, D), lambda i: (0, i, 0)),
            pl.BlockSpec((NC, r, DX), lambda i: (0, i, 0)),
            pl.BlockSpec((2 * D + DE, D), lambda i: (0, 0)),
            pl.BlockSpec((1, D), lambda i: (0, 0)),
            pl.BlockSpec((2 * D, D), lambda i: (0, 0)),
            pl.BlockSpec((1, D), lambda i: (0, 0)),
        ],
        out_specs=pl.BlockSpec((r, D), lambda i: (i, 0)),
        out_shape=jax.ShapeDtypeStruct((N_NODES, D), jnp.float32),
    )(x, a2p, a3xp, W_edge, b_edge2, W_node, b_node2)


def kernel(node_features, edge_index, edge_attr, W_edge, b_edge, W_node, b_node):
    ei32 = edge_index.astype(jnp.int32)
    first = ei32[0].reshape(NW, K, C)
    fs = ei32.reshape(2, NW, K, C).transpose(1, 2, 0, 3)
    attrx = jnp.concatenate(
        [edge_attr, jnp.ones((N_EDGES, DX - DE), jnp.float32)], axis=1,
    ).reshape(NW, K, C, DX)
    zeros_d = jnp.zeros((RPT, D), jnp.float32)
    zeros_x = jnp.zeros((RPT, DX), jnp.float32)
    a2p = _SC_A2(node_features, fs, zeros_d)
    a3xp = _SC_A3CNT(first, attrx, zeros_x)
    nodes = _combine(node_features, a2p, a3xp, W_edge,
                     b_edge.reshape(1, D), W_node, b_node.reshape(1, D))
    return (nodes, edge_index, edge_attr)


# speedup vs baseline: 1.4212x; 1.0242x over previous
import functools

import jax
import jax.numpy as jnp
from jax import lax
from jax.experimental import pallas as pl
from jax.experimental.pallas import tpu as pltpu
from jax.experimental.pallas import tpu_sc as plsc

N_NODES = 10000
N_EDGES = 320000
D = 128
DE = 16
NC = 2
NS = 16
NW = NC * NS
EPW = N_EDGES // NW
C = 80
K = EPW // C
N_PAD = 10240
RPT = N_PAD // NS

_MESH = plsc.VectorSubcoreMesh(core_axis_name="c", subcore_axis_name="s")


def _make_sc_a2():
    @functools.partial(
        pl.kernel,
        out_type=jax.ShapeDtypeStruct((NC, N_PAD, D), jnp.float32),
        mesh=_MESH,
        scratch_types=[
            [pltpu.VMEM((2, C), jnp.int32) for _ in range(3)],
            [pltpu.VMEM((C, D), jnp.float32) for _ in range(3)],
            pltpu.VMEM_SHARED((N_PAD, D), jnp.float32),
            [pltpu.SemaphoreType.DMA for _ in range(3)],
            [pltpu.SemaphoreType.DMA for _ in range(3)],
        ],
    )
    def sc_a2(x_hbm, fs_hbm, zeros_d_hbm, a2p_hbm,
              fs_v, rows_v, a2_sh, gsem, isem):
        cid = lax.axis_index("c")
        sid = lax.axis_index("s")
        wid = sid * NC + cid
        nbase = pl.multiple_of(sid * RPT, 8)
        pltpu.sync_copy(zeros_d_hbm, a2_sh.at[pl.ds(nbase, RPT)])
        pltpu.sync_copy(fs_hbm.at[wid, 0], fs_v[0])
        pltpu.async_copy(fs_hbm.at[wid, 1], fs_v[1], isem[1])
        pltpu.async_copy(fs_hbm.at[wid, 2], fs_v[2], isem[2])
        pltpu.async_copy(x_hbm.at[fs_v[0].at[1]], rows_v[0], gsem[0])
        pltpu.make_async_copy(fs_hbm.at[wid, 1], fs_v[1], isem[1]).wait()
        pltpu.async_copy(x_hbm.at[fs_v[1].at[1]], rows_v[1], gsem[1])
        plsc.subcore_barrier()

        def _step(j, b):
            pltpu.make_async_copy(x_hbm.at[fs_v[b].at[1]], rows_v[b],
                                  gsem[b]).wait()
            pltpu.sync_copy(rows_v[b], a2_sh.at[fs_v[b].at[0]], add=True)
            b2 = (b + 2) % 3
            pltpu.make_async_copy(fs_hbm.at[wid, j + 2], fs_v[b2],
                                  isem[b2]).wait()
            pltpu.async_copy(x_hbm.at[fs_v[b2].at[1]], rows_v[b2], gsem[b2])

            @pl.when(j + 3 < K)
            def _():
                pltpu.async_copy(fs_hbm.at[wid, j + 3], fs_v[b], isem[b])

        @pl.loop(0, (K - 2) // 3)
        def _triple(t):
            j = 3 * t
            _step(j, 0)
            _step(j + 1, 1)
            _step(j + 2, 2)

        pltpu.make_async_copy(x_hbm.at[fs_v[0].at[1]], rows_v[0],
                              gsem[0]).wait()
        pltpu.sync_copy(rows_v[0], a2_sh.at[fs_v[0].at[0]], add=True)
        pltpu.make_async_copy(x_hbm.at[fs_v[1].at[1]], rows_v[1],
                              gsem[1]).wait()
        pltpu.sync_copy(rows_v[1], a2_sh.at[fs_v[1].at[0]], add=True)

        plsc.subcore_barrier()
        pltpu.sync_copy(a2_sh.at[pl.ds(nbase, RPT)],
                        a2p_hbm.at[cid, pl.ds(nbase, RPT)])

    return sc_a2


DX = D


def _make_sc_a3cnt():
    @functools.partial(
        pl.kernel,
        out_type=jax.ShapeDtypeStruct((NC, N_PAD, DX), jnp.float32),
        mesh=_MESH,
        scratch_types=[
            pltpu.VMEM((K, C), jnp.int32),
            pltpu.VMEM((C, DX), jnp.float32),
            pltpu.VMEM((C, DX), jnp.float32),
            pltpu.VMEM_SHARED((N_PAD, DX), jnp.float32),
            pltpu.SemaphoreType.DMA,
            pltpu.SemaphoreType.DMA,
        ],
    )
    def sc_a3cnt(first_hbm, attrx_hbm, zeros_x_hbm, a3xp_hbm,
                 idx1_v, attrx0_v, attrx1_v, a3x_sh, sem0, sem1):
        cid = lax.axis_index("c")
        sid = lax.axis_index("s")
        wid = sid * NC + cid
        nbase = pl.multiple_of(sid * RPT, 8)
        pltpu.sync_copy(zeros_x_hbm, a3x_sh.at[pl.ds(nbase, RPT)])
        pltpu.sync_copy(first_hbm.at[wid], idx1_v)
        pltpu.async_copy(attrx_hbm.at[wid, 0], attrx0_v, sem0)
        plsc.subcore_barrier()

        @pl.loop(0, (K - 1) // 2)
        def _pair(p):
            j = 2 * p
            pltpu.make_async_copy(attrx_hbm.at[wid, j], attrx0_v, sem0).wait()
            pltpu.async_copy(attrx_hbm.at[wid, j + 1], attrx1_v, sem1)
            pltpu.sync_copy(attrx0_v, a3x_sh.at[idx1_v.at[j]], add=True)
            pltpu.make_async_copy(attrx_hbm.at[wid, j + 1], attrx1_v, sem1).wait()
            pltpu.async_copy(attrx_hbm.at[wid, j + 2], attrx0_v, sem0)
            pltpu.sync_copy(attrx1_v, a3x_sh.at[idx1_v.at[j + 1]], add=True)

        pltpu.make_async_copy(attrx_hbm.at[wid, K - 1], attrx0_v, sem0).wait()
        pltpu.sync_copy(attrx0_v, a3x_sh.at[idx1_v.at[K - 1]], add=True)

        plsc.subcore_barrier()
        pltpu.sync_copy(a3x_sh.at[pl.ds(nbase, RPT)],
                        a3xp_hbm.at[cid, pl.ds(nbase, RPT)])

    return sc_a3cnt


_SC_A2 = _make_sc_a2()
_SC_A3CNT = _make_sc_a3cnt()

_ROWS_PER_BLOCK = 400
_N_BLOCKS = N_NODES // _ROWS_PER_BLOCK


def _combine_body(x_ref, a2_ref, a3x_ref, we_ref, be_ref,
                  wn_ref, bn_ref, out_ref):
    f32 = jnp.float32
    x = x_ref[...]
    a2 = a2_ref[0] + a2_ref[1]
    a3x = a3x_ref[0] + a3x_ref[1]
    a3 = a3x[:, 0:DE]
    cnt = a3x[:, DE:DE + 1]
    ns = cnt * jnp.dot(x, we_ref[0:D, :], preferred_element_type=f32)
    ns = ns + jnp.dot(a2, we_ref[D:2 * D, :], preferred_element_type=f32)
    ns = ns + jnp.dot(a3, we_ref[2 * D:2 * D + DE, :], preferred_element_type=f32)
    ns = ns + cnt * be_ref[...]
    out = jnp.dot(x, wn_ref[0:D, :], preferred_element_type=f32)
    out = out + jnp.dot(ns, wn_ref[D:2 * D, :], preferred_element_type=f32)
    out_ref[...] = out + bn_ref[...]


def _combine(x, a2p, a3xp, W_edge, b_edge2, W_node, b_node2):
    r = _ROWS_PER_BLOCK
    return pl.pallas_call(
        _combine_body,
        grid=(_N_BLOCKS,),
        in_specs=[
            pl.BlockSpec((r, D), lambda i: (i, 0)),
            pl.BlockSpec((NC, r, D), lambda i: (0, i, 0)),
            pl.BlockSpec((NC, r, DX), lambda i: (0, i, 0)),
            pl.BlockSpec((2 * D + DE, D), lambda i: (0, 0)),
            pl.BlockSpec((1, D), lambda i: (0, 0)),
            pl.BlockSpec((2 * D, D), lambda i: (0, 0)),
            pl.BlockSpec((1, D), lambda i: (0, 0)),
        ],
        out_specs=pl.BlockSpec((r, D), lambda i: (i, 0)),
        out_shape=jax.ShapeDtypeStruct((N_NODES, D), jnp.float32),
    )(x, a2p, a3xp, W_edge, b_edge2, W_node, b_node2)


def kernel(node_features, edge_index, edge_attr, W_edge, b_edge, W_node, b_node):
    ei32 = edge_index.astype(jnp.int32)
    first = ei32[0].reshape(NW, K, C)
    fs = ei32.reshape(2, NW, K, C).transpose(1, 2, 0, 3)
    attrx = jnp.concatenate(
        [edge_attr, jnp.ones((N_EDGES, DX - DE), jnp.float32)], axis=1,
    ).reshape(NW, K, C, DX)
    zeros_d = jnp.zeros((RPT, D), jnp.float32)
    zeros_x = jnp.zeros((RPT, DX), jnp.float32)
    a2p = _SC_A2(node_features, fs, zeros_d)
    a3xp = _SC_A3CNT(first, attrx, zeros_x)
    nodes = _combine(node_features, a2p, a3xp, W_edge,
                     b_edge.reshape(1, D), W_node, b_node.reshape(1, D))
    return (nodes, edge_index, edge_attr)
